# Initial kernel scaffold; baseline (speedup 1.0000x reference)
import jax
import jax.numpy as jnp
from jax.experimental import pallas as pl


def kernel(x, edge_index, W1, b1, g1, be1, W2, b2, g2, be2):
    raise NotImplementedError("write your pallas kernel here")


# speedup vs baseline: 6.9368x; 6.9368x over previous
import functools

import jax
import jax.numpy as jnp
from jax import lax
from jax.experimental import pallas as pl
from jax.experimental.pallas import tpu as pltpu
from jax.experimental.pallas import tpu_sc as plsc

N = 10000
D = 256
E = 160000
EPS = 1e-5
HALF = 128

TILES = 16
EDGES_PER_TILE = E // TILES
CHUNK = 125
NCHUNK = EDGES_PER_TILE // CHUNK
IB = 16
NBLK = NCHUNK // IB
ROWS_MAIN = 624
ROWS_LAST = N - 15 * ROWS_MAIN

BM = 1000


def _mm_body(x_ref, w_ref, oa_ref, ob_ref):
    h = jnp.dot(x_ref[...], w_ref[...], preferred_element_type=jnp.float32)
    oa_ref[...] = h[:, :HALF]
    ob_ref[...] = h[:, HALF:]


def _mm(x, w):
    return pl.pallas_call(
        _mm_body,
        grid=(N // BM,),
        in_specs=[
            pl.BlockSpec((BM, D), lambda i: (i, 0)),
            pl.BlockSpec((D, D), lambda i: (0, 0)),
        ],
        out_specs=[
            pl.BlockSpec((BM, HALF), lambda i: (i, 0)),
            pl.BlockSpec((BM, HALF), lambda i: (i, 0)),
        ],
        out_shape=[jax.ShapeDtypeStruct((N, HALF), jnp.float32)] * 2,
    )(x, w)


def _stats_body(aa_ref, ab_ref, sa_ref, qa_ref, sb_ref, qb_ref):
    i = pl.program_id(0)

    @pl.when(i == 0)
    def _():
        sa_ref[...] = jnp.zeros_like(sa_ref)
        qa_ref[...] = jnp.zeros_like(qa_ref)
        sb_ref[...] = jnp.zeros_like(sb_ref)
        qb_ref[...] = jnp.zeros_like(qb_ref)

    aa = aa_ref[...]
    ab = ab_ref[...]
    sa_ref[...] += jnp.sum(aa, axis=0)[None, :]
    qa_ref[...] += jnp.sum(aa * aa, axis=0)[None, :]
    sb_ref[...] += jnp.sum(ab, axis=0)[None, :]
    qb_ref[...] += jnp.sum(ab * ab, axis=0)[None, :]


def _stats(aa, ab):
    return pl.pallas_call(
        _stats_body,
        grid=(N // BM,),
        in_specs=[
            pl.BlockSpec((BM, HALF), lambda i: (i, 0)),
            pl.BlockSpec((BM, HALF), lambda i: (i, 0)),
        ],
        out_specs=[pl.BlockSpec((8, HALF), lambda i: (0, 0))] * 4,
        out_shape=[jax.ShapeDtypeStruct((8, HALF), jnp.float32)] * 4,
    )(aa, ab)


def _affine(s_ref, q_ref, g, be):
    m = s_ref[0:1, :] * (1.0 / N)
    v = q_ref[0:1, :] * (1.0 / N) - m * m
    inv = lax.rsqrt(v + EPS)
    alpha = g * inv
    beta = be - m * alpha
    return alpha, beta


def _fused_body(aa_ref, ab_ref, sa_ref, qa_ref, sb_ref, qb_ref,
                g_ref, be_ref, w_ref, oa_ref, ob_ref):
    al_a, bt_a = _affine(sa_ref, qa_ref, g_ref[0:1, :HALF], be_ref[0:1, :HALF])
    al_b, bt_b = _affine(sb_ref, qb_ref, g_ref[0:1, HALF:], be_ref[0:1, HALF:])
    za = jnp.maximum(aa_ref[...] * al_a + bt_a, 0.0)
    zb = jnp.maximum(ab_ref[...] * al_b + bt_b, 0.0)
    h = (jnp.dot(za, w_ref[:HALF, :], preferred_element_type=jnp.float32)
         + jnp.dot(zb, w_ref[HALF:, :], preferred_element_type=jnp.float32))
    oa_ref[...] = h[:, :HALF]
    ob_ref[...] = h[:, HALF:]


def _fused(aa, ab, sa, qa, sb, qb, g, be, w):
    vec = pl.BlockSpec((8, HALF), lambda i: (0, 0))
    return pl.pallas_call(
        _fused_body,
        grid=(N // BM,),
        in_specs=[
            pl.BlockSpec((BM, HALF), lambda i: (i, 0)),
            pl.BlockSpec((BM, HALF), lambda i: (i, 0)),
            vec, vec, vec, vec,
            pl.BlockSpec((1, D), lambda i: (0, 0)),
            pl.BlockSpec((1, D), lambda i: (0, 0)),
            pl.BlockSpec((D, D), lambda i: (0, 0)),
        ],
        out_specs=[
            pl.BlockSpec((BM, HALF), lambda i: (i, 0)),
            pl.BlockSpec((BM, HALF), lambda i: (i, 0)),
        ],
        out_shape=[jax.ShapeDtypeStruct((N, HALF), jnp.float32)] * 2,
    )(aa, ab, sa, qa, sb, qb, g, be, w)


def _final_body(aa_ref, ab_ref, sa_ref, qa_ref, sb_ref, qb_ref,
                g_ref, be_ref, o_ref):
    al_a, bt_a = _affine(sa_ref, qa_ref, g_ref[0:1, :HALF], be_ref[0:1, :HALF])
    al_b, bt_b = _affine(sb_ref, qb_ref, g_ref[0:1, HALF:], be_ref[0:1, HALF:])
    o_ref[:, :HALF] = aa_ref[...] * al_a + bt_a
    o_ref[:, HALF:] = ab_ref[...] * al_b + bt_b


def _final(aa, ab, sa, qa, sb, qb, g, be):
    vec = pl.BlockSpec((8, HALF), lambda i: (0, 0))
    return pl.pallas_call(
        _final_body,
        grid=(N // BM,),
        in_specs=[
            pl.BlockSpec((BM, HALF), lambda i: (i, 0)),
            pl.BlockSpec((BM, HALF), lambda i: (i, 0)),
            vec, vec, vec, vec,
            pl.BlockSpec((1, D), lambda i: (0, 0)),
            pl.BlockSpec((1, D), lambda i: (0, 0)),
        ],
        out_specs=pl.BlockSpec((BM, D), lambda i: (i, 0)),
        out_shape=jax.ShapeDtypeStruct((N, D), jnp.float32),
    )(aa, ab, sa, qa, sb, qb, g, be)


def _scatter(ha, hb, src_r, dst_r, zb):
    mesh = plsc.VectorSubcoreMesh(core_axis_name="c", subcore_axis_name="s")

    @functools.partial(
        pl.kernel,
        mesh=mesh,
        out_type=[jax.ShapeDtypeStruct((N, HALF), jnp.float32)] * 2,
        scratch_types=[
            pltpu.VMEM((IB, CHUNK), jnp.int32),
            pltpu.VMEM((IB, CHUNK), jnp.int32),
            pltpu.VMEM((CHUNK, HALF), jnp.float32),
            pltpu.VMEM((CHUNK, HALF), jnp.float32),
            pltpu.VMEM_SHARED((N, HALF), jnp.float32),
            pltpu.SemaphoreType.DMA,
            pltpu.SemaphoreType.DMA,
        ],
    )
    def k(ha_hbm, hb_hbm, src_hbm, dst_hbm, zb_hbm, oa_hbm, ob_hbm,
          srcv, dstv, rows0, rows1, acc, sem0, sem1):
        c = lax.axis_index("c")
        s = lax.axis_index("s")
        r0 = s * ROWS_MAIN

        @pl.when(s < 15)
        def _():
            pltpu.sync_copy(zb_hbm.at[pl.ds(0, ROWS_MAIN)],
                            acc.at[pl.ds(r0, ROWS_MAIN)])

        @pl.when(s == 15)
        def _():
            pltpu.sync_copy(zb_hbm, acc.at[pl.ds(15 * ROWS_MAIN, ROWS_LAST)])

        plsc.subcore_barrier()

        bufs = (rows0, rows1)
        sems = (sem0, sem1)

        def run(h_hbm):
            def blk_body(blk, carry):
                j0 = blk * IB
                pltpu.sync_copy(src_hbm.at[s, pl.ds(j0, IB)], srcv)
                pltpu.sync_copy(dst_hbm.at[s, pl.ds(j0, IB)], dstv)
                pltpu.async_copy(h_hbm.at[srcv.at[0]], bufs[0], sems[0])

                def body(i, c2):
                    j = i * 2
                    for b in range(2):
                        jj = j + b
                        nb = (b + 1) % 2

                        @pl.when(jj + 1 < IB)
                        def _():
                            pltpu.async_copy(
                                h_hbm.at[srcv.at[jj + 1]], bufs[nb], sems[nb])

                        pltpu.make_async_copy(
                            h_hbm.at[srcv.at[jj]], bufs[b], sems[b]).wait()
                        pltpu.sync_copy(bufs[b], acc.at[dstv.at[jj]], add=True)
                    return c2

                lax.fori_loop(0, IB // 2, body, 0)
                return carry

            lax.fori_loop(0, NBLK, blk_body, 0)

        @pl.when(c == 0)
        def _():
            run(ha_hbm)

        @pl.when(c == 1)
        def _():
            run(hb_hbm)

        plsc.subcore_barrier()

        def drain(o_hbm):
            @pl.when(s < 15)
            def _():
                pltpu.sync_copy(acc.at[pl.ds(r0, ROWS_MAIN)],
                                o_hbm.at[pl.ds(r0, ROWS_MAIN)])

            @pl.when(s == 15)
            def _():
                pltpu.sync_copy(acc.at[pl.ds(15 * ROWS_MAIN, ROWS_LAST)],
                                o_hbm.at[pl.ds(15 * ROWS_MAIN, ROWS_LAST)])

        @pl.when(c == 0)
        def _():
            drain(oa_hbm)

        @pl.when(c == 1)
        def _():
            drain(ob_hbm)

    return k(ha, hb, src_r, dst_r, zb)


def kernel(x, edge_index, W1, b1, g1, be1, W2, b2, g2, be2):
    src_r = edge_index[0].reshape(TILES, NCHUNK, CHUNK)
    dst_r = edge_index[1].reshape(TILES, NCHUNK, CHUNK)
    zb = jnp.zeros((ROWS_LAST, HALF), jnp.float32)
    g1r = g1.reshape(1, D)
    be1r = be1.reshape(1, D)
    g2r = g2.reshape(1, D)
    be2r = be2.reshape(1, D)

    h1a, h1b = _mm(x, W1)
    a1a, a1b = _scatter(h1a, h1b, src_r, dst_r, zb)
    s1a, q1a, s1b, q1b = _stats(a1a, a1b)
    h2a, h2b = _fused(a1a, a1b, s1a, q1a, s1b, q1b, g1r, be1r, W2)
    a2a, a2b = _scatter(h2a, h2b, src_r, dst_r, zb)
    s2a, q2a, s2b, q2b = _stats(a2a, a2b)
    return _final(a2a, a2b, s2a, q2a, s2b, q2b, g2r, be2r)
